# Initial kernel scaffold; baseline (speedup 1.0000x reference)
import jax
import jax.numpy as jnp
from jax.experimental import pallas as pl


def kernel(x, y, t, p):
    raise NotImplementedError("write your pallas kernel here")


# speedup vs baseline: 33.9414x; 33.9414x over previous
import functools

import jax
import jax.numpy as jnp
from jax import lax
from jax.experimental import pallas as pl
from jax.experimental.pallas import tpu as pltpu
from jax.experimental.pallas import tpu_sc as plsc

N = 8388608
T = 8
H = 26
W = 40
BINS = 2 * T * H * W

NC = 2
NS = 16
NW = NC * NS
PER_W = N // NW
C = 16384
N_CHUNKS = PER_W // C
L = 16
VPC = C // L


def _hist_body(x_hbm, y_hbm, t_hbm, p_hbm, out_hbm,
               xv, yv, tv, pv, iv, ones_v, zv, hist):
    cid = lax.axis_index("c")
    sid = lax.axis_index("s")
    wid = sid * NC + cid

    one16 = jnp.full((L,), 1.0, dtype=jnp.float32)
    zero16 = jnp.zeros((L,), dtype=jnp.float32)

    def fill_ones(i, carry):
        ones_v[pl.ds(i * L, L)] = one16
        return carry

    lax.fori_loop(0, VPC, fill_ones, 0)

    @pl.when(sid == 0)
    def _():
        def fill_zero(i, carry):
            zv[pl.ds(i * L, L)] = zero16
            return carry

        lax.fori_loop(0, BINS // L, fill_zero, 0)
        pltpu.sync_copy(zv, hist)

    plsc.subcore_barrier()

    def chunk_body(j, carry):
        base = wid * PER_W + j * C
        pltpu.sync_copy(x_hbm.at[pl.ds(base, C)], xv)
        pltpu.sync_copy(y_hbm.at[pl.ds(base, C)], yv)
        pltpu.sync_copy(t_hbm.at[pl.ds(base, C)], tv)
        pltpu.sync_copy(p_hbm.at[pl.ds(base, C)], pv)

        def vec_body(i, carry2):
            s = pl.ds(i * L, L)
            xs = xv[s]
            ys = yv[s]
            ts = tv[s]
            ps = pv[s]
            xi = jnp.minimum(jnp.maximum(xs >> 3, 0), W - 1)
            yi = jnp.minimum(jnp.maximum(ys >> 3, 0), H - 1)
            ti = (ts * jnp.float32(T)).astype(jnp.int32)
            ti = jnp.minimum(jnp.maximum(ti, 0), T - 1)
            pi = jnp.minimum(jnp.maximum(ps, 0), 1)
            flat = ((pi << 3) + ti) * (H * W) + yi * W + xi
            iv[s] = flat
            return carry2

        lax.fori_loop(0, VPC, vec_body, 0)
        pltpu.sync_copy(ones_v, hist.at[iv], add=True)
        return carry

    lax.fori_loop(0, N_CHUNKS, chunk_body, 0)

    plsc.subcore_barrier()

    @pl.when(sid == 0)
    def _():
        pltpu.sync_copy(hist, out_hbm.at[cid])


@jax.jit
def _voxel_hist(x, y, t, p):
    mesh = plsc.VectorSubcoreMesh(
        core_axis_name="c", subcore_axis_name="s",
        num_cores=NC, num_subcores=NS,
    )
    partials = pl.kernel(
        _hist_body,
        out_type=jax.ShapeDtypeStruct((NC, BINS), jnp.float32),
        mesh=mesh,
        scratch_types=[
            pltpu.VMEM((C,), jnp.int32),
            pltpu.VMEM((C,), jnp.int32),
            pltpu.VMEM((C,), jnp.float32),
            pltpu.VMEM((C,), jnp.int32),
            pltpu.VMEM((C,), jnp.int32),
            pltpu.VMEM((C,), jnp.float32),
            pltpu.VMEM((BINS,), jnp.float32),
            pltpu.VMEM_SHARED((BINS,), jnp.float32),
        ],
    )(x, y, t, p)
    return partials.sum(axis=0).reshape(2 * T, H, W)


def kernel(x, y, t, p):
    return _voxel_hist(x, y, t, p)


# speedup vs baseline: 36.9972x; 1.0900x over previous
import functools

import jax
import jax.numpy as jnp
from jax import lax
from jax.experimental import pallas as pl
from jax.experimental.pallas import tpu as pltpu
from jax.experimental.pallas import tpu_sc as plsc

N = 8388608
T = 8
H = 26
W = 40
BINS = 2 * T * H * W

NC = 2
NS = 16
NW = NC * NS
PER_W = N // NW
C = 8192
N_CHUNKS = PER_W // C
L = 16
VPC = C // L
SLICE = BINS // NS


def _hist_body(x_hbm, y_hbm, t_hbm, p_hbm, out_hbm,
               xa, ya, ta, pa, xb, yb, tb, pb,
               histv, acc, tmp, slots, sem_a, sem_b):
    cid = lax.axis_index("c")
    sid = lax.axis_index("s")
    wid = sid * NC + cid
    ev_base = wid * PER_W

    zero16 = jnp.zeros((L,), dtype=jnp.float32)
    one16 = jnp.full((L,), 1.0, dtype=jnp.float32)

    def fill_zero(i, carry):
        histv[pl.ds(i * L, L)] = zero16
        return carry

    lax.fori_loop(0, BINS // L, fill_zero, 0)

    def start_loads(base, bufs, sem):
        xr, yr, tr, pr = bufs
        pltpu.async_copy(x_hbm.at[pl.ds(base, C)], xr, sem)
        pltpu.async_copy(y_hbm.at[pl.ds(base, C)], yr, sem)
        pltpu.async_copy(t_hbm.at[pl.ds(base, C)], tr, sem)
        pltpu.async_copy(p_hbm.at[pl.ds(base, C)], pr, sem)

    def wait_loads(bufs, sem):
        xr, yr, tr, pr = bufs
        pltpu.make_async_copy(x_hbm.at[pl.ds(0, C)], xr, sem).wait()
        pltpu.make_async_copy(y_hbm.at[pl.ds(0, C)], yr, sem).wait()
        pltpu.make_async_copy(t_hbm.at[pl.ds(0, C)], tr, sem).wait()
        pltpu.make_async_copy(p_hbm.at[pl.ds(0, C)], pr, sem).wait()

    bufs_a = (xa, ya, ta, pa)
    bufs_b = (xb, yb, tb, pb)

    def accumulate(bufs):
        xr, yr, tr, pr = bufs

        def vec_body(i, carry):
            s = pl.ds(i * L, L)
            xs = xr[s]
            ys = yr[s]
            ts = tr[s]
            ps = pr[s]
            xi = xs >> 3
            yi = jnp.minimum(ys >> 3, H - 1)
            ti = (ts * jnp.float32(T)).astype(jnp.int32)
            flat = ((ps << 3) + ti) * (H * W) + yi * W + xi
            plsc.addupdate_scatter(histv, [flat], one16)
            return carry

        lax.fori_loop(0, VPC, vec_body, 0)

    start_loads(ev_base, bufs_a, sem_a)

    def chunk_pair(jj, carry):
        start_loads(ev_base + (2 * jj + 1) * C, bufs_b, sem_b)
        wait_loads(bufs_a, sem_a)
        accumulate(bufs_a)

        @pl.when(jj + 1 < N_CHUNKS // 2)
        def _():
            start_loads(ev_base + (2 * jj + 2) * C, bufs_a, sem_a)

        wait_loads(bufs_b, sem_b)
        accumulate(bufs_b)
        return carry

    lax.fori_loop(0, N_CHUNKS // 2, chunk_pair, 0)

    pltpu.sync_copy(histv, slots.at[pl.ds(sid * BINS, BINS)])
    plsc.subcore_barrier()

    off = sid * SLICE
    pltpu.sync_copy(slots.at[pl.ds(off, SLICE)], acc)

    def red_body(k, carry):
        pltpu.sync_copy(slots.at[pl.ds(k * BINS + off, SLICE)], tmp)

        def add_body(i, carry2):
            s = pl.ds(i * L, L)
            acc[s] = acc[s] + tmp[s]
            return carry2

        lax.fori_loop(0, SLICE // L, add_body, 0)
        return carry

    lax.fori_loop(1, NS, red_body, 0)
    pltpu.sync_copy(acc, out_hbm.at[pl.ds(cid * BINS + off, SLICE)])


@jax.jit
def _voxel_hist(x, y, t, p):
    mesh = plsc.VectorSubcoreMesh(
        core_axis_name="c", subcore_axis_name="s",
        num_cores=NC, num_subcores=NS,
    )
    partials = pl.kernel(
        _hist_body,
        out_type=jax.ShapeDtypeStruct((NC * BINS,), jnp.float32),
        mesh=mesh,
        compiler_params=pltpu.CompilerParams(needs_layout_passes=False),
        scratch_types=[
            pltpu.VMEM((C,), jnp.int32),
            pltpu.VMEM((C,), jnp.int32),
            pltpu.VMEM((C,), jnp.float32),
            pltpu.VMEM((C,), jnp.int32),
            pltpu.VMEM((C,), jnp.int32),
            pltpu.VMEM((C,), jnp.int32),
            pltpu.VMEM((C,), jnp.float32),
            pltpu.VMEM((C,), jnp.int32),
            pltpu.VMEM((BINS,), jnp.float32),
            pltpu.VMEM((SLICE,), jnp.float32),
            pltpu.VMEM((SLICE,), jnp.float32),
            pltpu.VMEM_SHARED((NS * BINS,), jnp.float32),
            pltpu.SemaphoreType.DMA,
            pltpu.SemaphoreType.DMA,
        ],
    )(x, y, t, p)
    return partials.reshape(NC, BINS).sum(axis=0).reshape(2 * T, H, W)


def kernel(x, y, t, p):
    return _voxel_hist(x, y, t, p)


# speedup vs baseline: 89.2627x; 2.4127x over previous
import functools

import jax
import jax.numpy as jnp
from jax import lax
from jax.experimental import pallas as pl
from jax.experimental.pallas import tpu as pltpu
from jax.experimental.pallas import tpu_sc as plsc

N = 8388608
T = 8
H = 26
W = 40
BINS = 2 * T * H * W

NC = 2
NS = 16
NW = NC * NS
PER_W = N // NW
C = 8192
N_CHUNKS = PER_W // C
L = 16
VPC = C // L
SLICE = BINS // NS


def _hist_body(x_hbm, y_hbm, t_hbm, p_hbm, out_hbm,
               xa, ya, ta, pa, xb, yb, tb, pb,
               histv, acc, tmp, slots, sem_a, sem_b):
    cid = lax.axis_index("c")
    sid = lax.axis_index("s")
    wid = sid * NC + cid
    ev_base = wid * PER_W

    zero16 = jnp.zeros((L,), dtype=jnp.float32)
    one16 = jnp.full((L,), 1.0, dtype=jnp.float32)

    def fill_zero(i, carry):
        histv[pl.ds(i * L, L)] = zero16
        return carry

    lax.fori_loop(0, BINS // L, fill_zero, 0)

    def start_loads(base, bufs, sem):
        xr, yr, tr, pr = bufs
        pltpu.async_copy(x_hbm.at[pl.ds(base, C)], xr, sem)
        pltpu.async_copy(y_hbm.at[pl.ds(base, C)], yr, sem)
        pltpu.async_copy(t_hbm.at[pl.ds(base, C)], tr, sem)
        pltpu.async_copy(p_hbm.at[pl.ds(base, C)], pr, sem)

    def wait_loads(bufs, sem):
        xr, yr, tr, pr = bufs
        pltpu.make_async_copy(x_hbm.at[pl.ds(0, C)], xr, sem).wait()
        pltpu.make_async_copy(y_hbm.at[pl.ds(0, C)], yr, sem).wait()
        pltpu.make_async_copy(t_hbm.at[pl.ds(0, C)], tr, sem).wait()
        pltpu.make_async_copy(p_hbm.at[pl.ds(0, C)], pr, sem).wait()

    bufs_a = (xa, ya, ta, pa)
    bufs_b = (xb, yb, tb, pb)

    def accumulate(bufs):
        xr, yr, tr, pr = bufs

        @plsc.parallel_loop(0, VPC, unroll=8)
        def vec_body(i):
            s = pl.ds(i * L, L)
            xs = xr[s]
            ys = yr[s]
            ts = tr[s]
            ps = pr[s]
            xi = xs >> 3
            yi = jnp.minimum(ys >> 3, H - 1)
            ti = (ts * jnp.float32(T)).astype(jnp.int32)
            flat = ((ps << 3) + ti) * (H * W) + yi * W + xi
            plsc.addupdate_scatter(histv, [flat], one16)

    start_loads(ev_base, bufs_a, sem_a)

    def chunk_pair(jj, carry):
        start_loads(ev_base + (2 * jj + 1) * C, bufs_b, sem_b)
        wait_loads(bufs_a, sem_a)
        accumulate(bufs_a)

        @pl.when(jj + 1 < N_CHUNKS // 2)
        def _():
            start_loads(ev_base + (2 * jj + 2) * C, bufs_a, sem_a)

        wait_loads(bufs_b, sem_b)
        accumulate(bufs_b)
        return carry

    lax.fori_loop(0, N_CHUNKS // 2, chunk_pair, 0)

    pltpu.sync_copy(histv, slots.at[pl.ds(sid * BINS, BINS)])
    plsc.subcore_barrier()

    off = sid * SLICE
    pltpu.sync_copy(slots.at[pl.ds(off, SLICE)], acc)

    def red_body(k, carry):
        pltpu.sync_copy(slots.at[pl.ds(k * BINS + off, SLICE)], tmp)

        def add_body(i, carry2):
            s = pl.ds(i * L, L)
            acc[s] = acc[s] + tmp[s]
            return carry2

        lax.fori_loop(0, SLICE // L, add_body, 0)
        return carry

    lax.fori_loop(1, NS, red_body, 0)
    pltpu.sync_copy(acc, out_hbm.at[pl.ds(cid * BINS + off, SLICE)])


@jax.jit
def _voxel_hist(x, y, t, p):
    mesh = plsc.VectorSubcoreMesh(
        core_axis_name="c", subcore_axis_name="s",
        num_cores=NC, num_subcores=NS,
    )
    partials = pl.kernel(
        _hist_body,
        out_type=jax.ShapeDtypeStruct((NC * BINS,), jnp.float32),
        mesh=mesh,
        compiler_params=pltpu.CompilerParams(needs_layout_passes=False),
        scratch_types=[
            pltpu.VMEM((C,), jnp.int32),
            pltpu.VMEM((C,), jnp.int32),
            pltpu.VMEM((C,), jnp.float32),
            pltpu.VMEM((C,), jnp.int32),
            pltpu.VMEM((C,), jnp.int32),
            pltpu.VMEM((C,), jnp.int32),
            pltpu.VMEM((C,), jnp.float32),
            pltpu.VMEM((C,), jnp.int32),
            pltpu.VMEM((BINS,), jnp.float32),
            pltpu.VMEM((SLICE,), jnp.float32),
            pltpu.VMEM((SLICE,), jnp.float32),
            pltpu.VMEM_SHARED((NS * BINS,), jnp.float32),
            pltpu.SemaphoreType.DMA,
            pltpu.SemaphoreType.DMA,
        ],
    )(x, y, t, p)
    return partials.reshape(NC, BINS).sum(axis=0).reshape(2 * T, H, W)


def kernel(x, y, t, p):
    return _voxel_hist(x, y, t, p)
